# Initial kernel scaffold; baseline (speedup 1.0000x reference)
import jax
import jax.numpy as jnp
from jax.experimental import pallas as pl


def kernel(x, edge_index, W10, b10, W11, b11, W12, b12, w1, b1, fc0_W, fc0_b, fc1_W, fc1_b):
    raise NotImplementedError("write your pallas kernel here")


# speedup vs baseline: 14.8208x; 14.8208x over previous
import functools

import jax
import jax.numpy as jnp
from jax import lax
from jax.experimental import pallas as pl
from jax.experimental.pallas import tpu as pltpu
from jax.experimental.pallas import tpu_sc as plsc

_N = 10000
_E = 320000
_H = 128
_FCW = 1024
_NCLS = 16

_NSC = 2
_NSUB = 16
_NW = _NSC * _NSUB
_EPW = _E // _NW
_K = 125
_CH = _EPW // _K
_ACCR = 10240
_RT = _ACCR // _NSUB
_RB = 80

_sc_mesh = plsc.VectorSubcoreMesh(
    core_axis_name="c", subcore_axis_name="s", num_cores=_NSC, num_subcores=_NSUB
)


@functools.partial(
    pl.kernel,
    out_type=jax.ShapeDtypeStruct((_NSC, _N, _H), jnp.float32),
    mesh=_sc_mesh,
    scratch_types=[
        pltpu.VMEM_SHARED((_ACCR, _H), jnp.float32),
        pltpu.VMEM((_CH, _K), jnp.int32),
        pltpu.VMEM((_CH, _K), jnp.int32),
        pltpu.VMEM((_K, _H), jnp.float32),
        pltpu.VMEM((_RB, _H), jnp.float32),
        pltpu.SemaphoreType.DMA,
    ],
)
def _sc_aggregate(g_hbm, src_hbm, dst_hbm, out_hbm, acc, src_v, dst_v, rows_v,
                  stage_v, sem):
    c = lax.axis_index("c")
    s = lax.axis_index("s")
    w = c * _NSUB + s
    pltpu.sync_copy(src_hbm.at[w], src_v)
    pltpu.sync_copy(dst_hbm.at[w], dst_v)
    base = s * _RT
    nblk = jnp.where(s == _NSUB - 1, 5, 8)

    @pl.when(c == 0)
    def _():
        def seed(r, carry):
            pltpu.sync_copy(g_hbm.at[pl.ds(base + r * _RB, _RB)], stage_v)
            pltpu.sync_copy(stage_v, acc.at[pl.ds(base + r * _RB, _RB)])
            return carry

        lax.fori_loop(0, nblk, seed, 0)

    @pl.when(c != 0)
    def _():
        zeros16 = jnp.zeros((16,), jnp.float32)

        def zfill(i, carry):
            for jj in range(_H // 16):
                stage_v[i, pl.ds(jj * 16, 16)] = zeros16
            return carry

        lax.fori_loop(0, _RB, zfill, 0)

        def zinit(r, carry):
            pltpu.sync_copy(stage_v, acc.at[pl.ds(base + r * _RB, _RB)])
            return carry

        lax.fori_loop(0, nblk, zinit, 0)

    plsc.subcore_barrier()

    def step(j, carry):
        pltpu.async_copy(g_hbm.at[src_v.at[j]], rows_v, sem).wait()
        pltpu.sync_copy(rows_v, acc.at[dst_v.at[j]], add=True)
        return carry

    lax.fori_loop(0, _CH, step, 0)
    plsc.subcore_barrier()

    def wb(r, carry):
        pltpu.sync_copy(acc.at[pl.ds(base + r * _RB, _RB)], stage_v)
        pltpu.sync_copy(stage_v, out_hbm.at[c, pl.ds(base + r * _RB, _RB)])
        return carry

    lax.fori_loop(0, nblk, wb, 0)


_RBLK = 1000


def _tc_a(x, W10, degpart):
    def body(x_ref, w_ref, dp_ref, g_ref, dinv_ref):
        deg = dp_ref[0, :, 0:1] + dp_ref[1, :, 0:1]
        dv = lax.rsqrt(deg)
        g_ref[...] = jnp.dot(x_ref[...], w_ref[...],
                             preferred_element_type=jnp.float32) * dv
        dinv_ref[...] = dv

    return pl.pallas_call(
        body,
        grid=(_N // _RBLK,),
        in_specs=[
            pl.BlockSpec((_RBLK, _H), lambda i: (i, 0)),
            pl.BlockSpec((_H, _H), lambda i: (0, 0)),
            pl.BlockSpec((_NSC, _RBLK, _H), lambda i: (0, i, 0)),
        ],
        out_specs=[
            pl.BlockSpec((_RBLK, _H), lambda i: (i, 0)),
            pl.BlockSpec((_RBLK, 1), lambda i: (i, 0)),
        ],
        out_shape=[
            jax.ShapeDtypeStruct((_N, _H), jnp.float32),
            jax.ShapeDtypeStruct((_N, 1), jnp.float32),
        ],
    )(x, W10, degpart)


def _tc_b(P1, dinv, b10, w1, b1, W11):
    def body(p_ref, dinv_ref, b10_ref, w1_ref, b1_ref, w11_ref, h1_ref, g2_ref):
        dv = dinv_ref[...]
        h1 = jnp.maximum((p_ref[0] + p_ref[1]) * dv + b10_ref[...], 0.0)
        h11 = jnp.maximum(
            jnp.dot(h1, w1_ref[...], preferred_element_type=jnp.float32)
            + b1_ref[...], 0.0)
        g2_ref[...] = jnp.dot(h11, w11_ref[...],
                              preferred_element_type=jnp.float32) * dv
        h1_ref[...] = h1

    return pl.pallas_call(
        body,
        grid=(_N // _RBLK,),
        in_specs=[
            pl.BlockSpec((_NSC, _RBLK, _H), lambda i: (0, i, 0)),
            pl.BlockSpec((_RBLK, 1), lambda i: (i, 0)),
            pl.BlockSpec((1, _H), lambda i: (0, 0)),
            pl.BlockSpec((_H, _H), lambda i: (0, 0)),
            pl.BlockSpec((_RBLK, _H), lambda i: (i, 0)),
            pl.BlockSpec((_H, _H), lambda i: (0, 0)),
        ],
        out_specs=[
            pl.BlockSpec((_RBLK, _H), lambda i: (i, 0)),
            pl.BlockSpec((_RBLK, _H), lambda i: (i, 0)),
        ],
        out_shape=[
            jax.ShapeDtypeStruct((_N, _H), jnp.float32),
            jax.ShapeDtypeStruct((_N, _H), jnp.float32),
        ],
    )(P1, dinv, b10, w1, b1, W11)


def _tc_c(P2, dinv, b11, W12):
    def body(p_ref, dinv_ref, b11_ref, w12_ref, g3_ref):
        dv = dinv_ref[...]
        h2 = (p_ref[0] + p_ref[1]) * dv + b11_ref[...]
        g3_ref[...] = jnp.dot(h2, w12_ref[...],
                              preferred_element_type=jnp.float32) * dv

    return pl.pallas_call(
        body,
        grid=(_N // _RBLK,),
        in_specs=[
            pl.BlockSpec((_NSC, _RBLK, _H), lambda i: (0, i, 0)),
            pl.BlockSpec((_RBLK, 1), lambda i: (i, 0)),
            pl.BlockSpec((1, _H), lambda i: (0, 0)),
            pl.BlockSpec((_H, _H), lambda i: (0, 0)),
        ],
        out_specs=pl.BlockSpec((_RBLK, _H), lambda i: (i, 0)),
        out_shape=jax.ShapeDtypeStruct((_N, _H), jnp.float32),
    )(P2, dinv, b11, W12)


def _tc_d(P3, dinv, b12, h1, fc0_W, fc0_b, fc1_W, fc1_b):
    def body(p_ref, dinv_ref, b12_ref, h1_ref, fc0w_ref, fc0b_ref, fc1w_ref,
             fc1b_ref, out_ref):
        h2 = jnp.maximum((p_ref[0] + p_ref[1]) * dinv_ref[...] + b12_ref[...],
                         0.0)
        h = h1_ref[...] + h2
        t = jnp.maximum(
            jnp.dot(h, fc0w_ref[...], preferred_element_type=jnp.float32)
            + fc0b_ref[...], 0.0)
        out_ref[...] = (
            jnp.dot(t, fc1w_ref[...], preferred_element_type=jnp.float32)
            + fc1b_ref[...])

    return pl.pallas_call(
        body,
        grid=(_N // _RBLK,),
        in_specs=[
            pl.BlockSpec((_NSC, _RBLK, _H), lambda i: (0, i, 0)),
            pl.BlockSpec((_RBLK, 1), lambda i: (i, 0)),
            pl.BlockSpec((1, _H), lambda i: (0, 0)),
            pl.BlockSpec((_RBLK, _H), lambda i: (i, 0)),
            pl.BlockSpec((_H, _FCW), lambda i: (0, 0)),
            pl.BlockSpec((1, _FCW), lambda i: (0, 0)),
            pl.BlockSpec((_FCW, _NCLS), lambda i: (0, 0)),
            pl.BlockSpec((1, _NCLS), lambda i: (0, 0)),
        ],
        out_specs=pl.BlockSpec((_RBLK, _NCLS), lambda i: (i, 0)),
        out_shape=jax.ShapeDtypeStruct((_N, _NCLS), jnp.float32),
    )(P3, dinv, b12, h1, fc0_W, fc0_b, fc1_W, fc1_b)


def kernel(x, edge_index, W10, b10, W11, b11, W12, b12, w1, b1, fc0_W, fc0_b,
           fc1_W, fc1_b):
    src = edge_index[0].reshape(_NW, _CH, _K)
    dst = edge_index[1].reshape(_NW, _CH, _K)

    ones = jnp.ones((_N, _H), jnp.float32)
    degpart = _sc_aggregate(ones, src, dst)
    g1, dinv = _tc_a(x, W10, degpart)
    P1 = _sc_aggregate(g1, src, dst)
    h1, g2 = _tc_b(P1, dinv, b10.reshape(1, -1), w1, b1, W11)
    P2 = _sc_aggregate(g2, src, dst)
    g3 = _tc_c(P2, dinv, b11.reshape(1, -1), W12)
    P3 = _sc_aggregate(g3, src, dst)
    out = _tc_d(P3, dinv, b12.reshape(1, -1), h1, fc0_W, fc0_b.reshape(1, -1),
                fc1_W, fc1_b.reshape(1, -1))
    return out


# speedup vs baseline: 18.2215x; 1.2295x over previous
import functools

import jax
import jax.numpy as jnp
from jax import lax
from jax.experimental import pallas as pl
from jax.experimental.pallas import tpu as pltpu
from jax.experimental.pallas import tpu_sc as plsc

_N = 10000
_E = 320000
_H = 128
_FCW = 1024
_NCLS = 16

_NSC = 2
_NSUB = 16
_NW = _NSC * _NSUB
_EPW = _E // _NW
_K = 40
_CH = _EPW // _K
_NB = 5
_OWN = 624
_RB = 16

_sc_mesh = plsc.VectorSubcoreMesh(
    core_axis_name="c", subcore_axis_name="s", num_cores=_NSC, num_subcores=_NSUB
)


@functools.partial(
    pl.kernel,
    out_type=jax.ShapeDtypeStruct((_NSC, _N, _H), jnp.float32),
    mesh=_sc_mesh,
    scratch_types=[
        pltpu.VMEM_SHARED((_N, _H), jnp.float32),
        pltpu.VMEM((_EPW,), jnp.int32),
        pltpu.VMEM((1, _K), jnp.int32),
        pltpu.VMEM((1, _K), jnp.int32),
        pltpu.VMEM((1, _K), jnp.int32),
        pltpu.VMEM((1, _K), jnp.int32),
        pltpu.VMEM((1, _K), jnp.int32),
        pltpu.VMEM((_K, _H), jnp.float32),
        pltpu.VMEM((_K, _H), jnp.float32),
        pltpu.VMEM((_K, _H), jnp.float32),
        pltpu.VMEM((_K, _H), jnp.float32),
        pltpu.VMEM((_K, _H), jnp.float32),
        pltpu.SemaphoreType.DMA,
        pltpu.SemaphoreType.DMA,
        pltpu.SemaphoreType.DMA,
        pltpu.SemaphoreType.DMA,
        pltpu.SemaphoreType.DMA,
        pltpu.SemaphoreType.DMA,
        pltpu.SemaphoreType.DMA,
        pltpu.SemaphoreType.DMA,
        pltpu.SemaphoreType.DMA,
        pltpu.SemaphoreType.DMA,
        pltpu.SemaphoreType.DMA,
        pltpu.SemaphoreType.DMA,
        pltpu.SemaphoreType.DMA,
        pltpu.SemaphoreType.DMA,
        pltpu.SemaphoreType.DMA,
    ],
)
def _sc_aggregate(g_hbm, src_hbm, dst_hbm, out_hbm, acc, src_v,
                  i0, i1, i2, i3, i4, r0, r1, r2, r3, r4,
                  is0, is1, is2, is3, is4, gs0, gs1, gs2, gs3, gs4,
                  ss0, ss1, ss2, ss3, ss4):
    c = lax.axis_index("c")
    s = lax.axis_index("s")
    w = c * _NSUB + s
    ebase = w * _EPW
    pltpu.sync_copy(src_hbm.at[pl.ds(ebase, _EPW)], src_v)
    base = s * _OWN
    nblk = jnp.where(s == _NSUB - 1, 40, 39)
    stage = r0.at[pl.ds(0, _RB)]

    @pl.when(c == 0)
    def _():
        def seed(r, carry):
            pltpu.sync_copy(g_hbm.at[pl.ds(base + r * _RB, _RB)], stage)
            pltpu.sync_copy(stage, acc.at[pl.ds(base + r * _RB, _RB)])
            return carry

        lax.fori_loop(0, nblk, seed, 0)

    @pl.when(c != 0)
    def _():
        zeros16 = jnp.zeros((16,), jnp.float32)

        def zfill(i, carry):
            for jj in range(_H // 16):
                r0[i, pl.ds(jj * 16, 16)] = zeros16
            return carry

        lax.fori_loop(0, _RB, zfill, 0)

        def zinit(r, carry):
            pltpu.sync_copy(stage, acc.at[pl.ds(base + r * _RB, _RB)])
            return carry

        lax.fori_loop(0, nblk, zinit, 0)

    plsc.subcore_barrier()

    idx = (i0, i1, i2, i3, i4)
    rows = (r0, r1, r2, r3, r4)
    iss = (is0, is1, is2, is3, is4)
    gs = (gs0, gs1, gs2, gs3, gs4)
    ss = (ss0, ss1, ss2, ss3, ss4)

    def fetch_dst(b, j):
        pltpu.async_copy(dst_hbm.at[pl.ds(ebase + j * _K, _K)],
                         idx[b].at[0], iss[b])

    def wait_dst(b, j):
        pltpu.make_async_copy(dst_hbm.at[pl.ds(ebase + j * _K, _K)],
                              idx[b].at[0], iss[b]).wait()

    def gather(b, j):
        pltpu.async_copy(g_hbm.at[src_v.at[pl.ds(j * _K, _K)]], rows[b], gs[b])

    def wait_gather(b, j):
        pltpu.make_async_copy(g_hbm.at[src_v.at[pl.ds(j * _K, _K)]],
                              rows[b], gs[b]).wait()

    def scatter(b):
        pltpu.async_copy(rows[b], acc.at[idx[b].at[0]], ss[b], add=True)

    def wait_scatter(b):
        pltpu.make_async_copy(rows[b], acc.at[idx[b].at[0]], ss[b]).wait()

    for b in range(_NB):
        fetch_dst(b, b)
        gather(b, b)

    def pipe(jj, carry):
        j0 = jj * _NB
        for b in range(_NB):
            wait_gather(b, j0 + b)
            wait_dst(b, j0 + b)
            scatter(b)
        for b in range(_NB):
            jn = lax.rem(j0 + b + _NB, _CH)
            wait_scatter(b)
            fetch_dst(b, jn)
            gather(b, jn)
        return carry

    lax.fori_loop(0, _CH // _NB, pipe, 0)
    for b in range(_NB):
        wait_gather(b, b)
        wait_dst(b, b)
    plsc.subcore_barrier()

    def wb(r, carry):
        pltpu.sync_copy(acc.at[pl.ds(base + r * _RB, _RB)], stage)
        pltpu.sync_copy(stage, out_hbm.at[c, pl.ds(base + r * _RB, _RB)])
        return carry

    lax.fori_loop(0, nblk, wb, 0)


_RBLK = 1000


def _tc_a(x, W10, degpart):
    def body(x_ref, w_ref, dp_ref, g_ref, dinv_ref):
        dv = lax.rsqrt(dp_ref[0, :, 0:1] + dp_ref[1, :, 0:1])
        g_ref[...] = jnp.dot(x_ref[...], w_ref[...],
                             preferred_element_type=jnp.float32) * dv
        dinv_ref[...] = dv

    return pl.pallas_call(
        body,
        grid=(_N // _RBLK,),
        in_specs=[
            pl.BlockSpec((_RBLK, _H), lambda i: (i, 0)),
            pl.BlockSpec((_H, _H), lambda i: (0, 0)),
            pl.BlockSpec((_NSC, _RBLK, _H), lambda i: (0, i, 0)),
        ],
        out_specs=[
            pl.BlockSpec((_RBLK, _H), lambda i: (i, 0)),
            pl.BlockSpec((_RBLK, 1), lambda i: (i, 0)),
        ],
        out_shape=[
            jax.ShapeDtypeStruct((_N, _H), jnp.float32),
            jax.ShapeDtypeStruct((_N, 1), jnp.float32),
        ],
    )(x, W10, degpart)


def _tc_b(P1, dinv, b10, w1, b1, W11):
    def body(p_ref, dinv_ref, b10_ref, w1_ref, b1_ref, w11_ref, h1_ref, g2_ref):
        dv = dinv_ref[...]
        h1 = jnp.maximum((p_ref[0] + p_ref[1]) * dv + b10_ref[...], 0.0)
        h11 = jnp.maximum(
            jnp.dot(h1, w1_ref[...], preferred_element_type=jnp.float32)
            + b1_ref[...], 0.0)
        g2_ref[...] = jnp.dot(h11, w11_ref[...],
                              preferred_element_type=jnp.float32) * dv
        h1_ref[...] = h1

    return pl.pallas_call(
        body,
        grid=(_N // _RBLK,),
        in_specs=[
            pl.BlockSpec((_NSC, _RBLK, _H), lambda i: (0, i, 0)),
            pl.BlockSpec((_RBLK, 1), lambda i: (i, 0)),
            pl.BlockSpec((1, _H), lambda i: (0, 0)),
            pl.BlockSpec((_H, _H), lambda i: (0, 0)),
            pl.BlockSpec((_RBLK, _H), lambda i: (i, 0)),
            pl.BlockSpec((_H, _H), lambda i: (0, 0)),
        ],
        out_specs=[
            pl.BlockSpec((_RBLK, _H), lambda i: (i, 0)),
            pl.BlockSpec((_RBLK, _H), lambda i: (i, 0)),
        ],
        out_shape=[
            jax.ShapeDtypeStruct((_N, _H), jnp.float32),
            jax.ShapeDtypeStruct((_N, _H), jnp.float32),
        ],
    )(P1, dinv, b10, w1, b1, W11)


def _tc_c(P2, dinv, b11, W12):
    def body(p_ref, dinv_ref, b11_ref, w12_ref, g3_ref):
        dv = dinv_ref[...]
        h2 = (p_ref[0] + p_ref[1]) * dv + b11_ref[...]
        g3_ref[...] = jnp.dot(h2, w12_ref[...],
                              preferred_element_type=jnp.float32) * dv

    return pl.pallas_call(
        body,
        grid=(_N // _RBLK,),
        in_specs=[
            pl.BlockSpec((_NSC, _RBLK, _H), lambda i: (0, i, 0)),
            pl.BlockSpec((_RBLK, 1), lambda i: (i, 0)),
            pl.BlockSpec((1, _H), lambda i: (0, 0)),
            pl.BlockSpec((_H, _H), lambda i: (0, 0)),
        ],
        out_specs=pl.BlockSpec((_RBLK, _H), lambda i: (i, 0)),
        out_shape=jax.ShapeDtypeStruct((_N, _H), jnp.float32),
    )(P2, dinv, b11, W12)


def _tc_d(P3, dinv, b12, h1, fc0_W, fc0_b, fc1_W, fc1_b):
    def body(p_ref, dinv_ref, b12_ref, h1_ref, fc0w_ref, fc0b_ref, fc1w_ref,
             fc1b_ref, out_ref):
        h2 = jnp.maximum((p_ref[0] + p_ref[1]) * dinv_ref[...] + b12_ref[...],
                         0.0)
        h = h1_ref[...] + h2
        t = jnp.maximum(
            jnp.dot(h, fc0w_ref[...], preferred_element_type=jnp.float32)
            + fc0b_ref[...], 0.0)
        out_ref[...] = (
            jnp.dot(t, fc1w_ref[...], preferred_element_type=jnp.float32)
            + fc1b_ref[...])

    return pl.pallas_call(
        body,
        grid=(_N // _RBLK,),
        in_specs=[
            pl.BlockSpec((_NSC, _RBLK, _H), lambda i: (0, i, 0)),
            pl.BlockSpec((_RBLK, 1), lambda i: (i, 0)),
            pl.BlockSpec((1, _H), lambda i: (0, 0)),
            pl.BlockSpec((_RBLK, _H), lambda i: (i, 0)),
            pl.BlockSpec((_H, _FCW), lambda i: (0, 0)),
            pl.BlockSpec((1, _FCW), lambda i: (0, 0)),
            pl.BlockSpec((_FCW, _NCLS), lambda i: (0, 0)),
            pl.BlockSpec((1, _NCLS), lambda i: (0, 0)),
        ],
        out_specs=pl.BlockSpec((_RBLK, _NCLS), lambda i: (i, 0)),
        out_shape=jax.ShapeDtypeStruct((_N, _NCLS), jnp.float32),
    )(P3, dinv, b12, h1, fc0_W, fc0_b, fc1_W, fc1_b)


def kernel(x, edge_index, W10, b10, W11, b11, W12, b12, w1, b1, fc0_W, fc0_b,
           fc1_W, fc1_b):
    src = edge_index[0]
    dst = edge_index[1]

    ones = jnp.ones((_N, _H), jnp.float32)
    degpart = _sc_aggregate(ones, src, dst)
    g1, dinv = _tc_a(x, W10, degpart)
    P1 = _sc_aggregate(g1, src, dst)
    h1, g2 = _tc_b(P1, dinv, b10.reshape(1, -1), w1, b1, W11)
    P2 = _sc_aggregate(g2, src, dst)
    g3 = _tc_c(P2, dinv, b11.reshape(1, -1), W12)
    P3 = _sc_aggregate(g3, src, dst)
    out = _tc_d(P3, dinv, b12.reshape(1, -1), h1, fc0_W, fc0_b.reshape(1, -1),
                fc1_W, fc1_b.reshape(1, -1))
    return out


# speedup vs baseline: 21.6427x; 1.1878x over previous
import functools

import jax
import jax.numpy as jnp
from jax import lax
from jax.experimental import pallas as pl
from jax.experimental.pallas import tpu as pltpu
from jax.experimental.pallas import tpu_sc as plsc

_N = 10000
_E = 320000
_H = 128
_FCW = 1024
_NCLS = 16

_NSC = 2
_NSUB = 16
_NW = _NSC * _NSUB
_EPW = _E // _NW
_K = 40
_CH = _EPW // _K
_NB = 5
_OWN = 624
_OWNL = _N - (_NSUB - 1) * _OWN

_sc_mesh = plsc.VectorSubcoreMesh(
    core_axis_name="c", subcore_axis_name="s", num_cores=_NSC, num_subcores=_NSUB
)


@functools.partial(
    pl.kernel,
    out_type=jax.ShapeDtypeStruct((_NSC, _N, _H), jnp.float32),
    mesh=_sc_mesh,
    scratch_types=[
        pltpu.VMEM_SHARED((_N, _H), jnp.float32),
        pltpu.VMEM((_EPW,), jnp.int32),
        pltpu.VMEM((1, _K), jnp.int32),
        pltpu.VMEM((1, _K), jnp.int32),
        pltpu.VMEM((1, _K), jnp.int32),
        pltpu.VMEM((1, _K), jnp.int32),
        pltpu.VMEM((1, _K), jnp.int32),
        pltpu.VMEM((_K, _H), jnp.float32),
        pltpu.VMEM((_K, _H), jnp.float32),
        pltpu.VMEM((_K, _H), jnp.float32),
        pltpu.VMEM((_K, _H), jnp.float32),
        pltpu.VMEM((_K, _H), jnp.float32),
        pltpu.SemaphoreType.DMA,
        pltpu.SemaphoreType.DMA,
        pltpu.SemaphoreType.DMA,
        pltpu.SemaphoreType.DMA,
        pltpu.SemaphoreType.DMA,
        pltpu.SemaphoreType.DMA,
        pltpu.SemaphoreType.DMA,
        pltpu.SemaphoreType.DMA,
        pltpu.SemaphoreType.DMA,
        pltpu.SemaphoreType.DMA,
        pltpu.SemaphoreType.DMA,
        pltpu.SemaphoreType.DMA,
        pltpu.SemaphoreType.DMA,
        pltpu.SemaphoreType.DMA,
        pltpu.SemaphoreType.DMA,
    ],
)
def _sc_aggregate(g_hbm, src_hbm, dst_hbm, out_hbm, acc, src_v,
                  i0, i1, i2, i3, i4, r0, r1, r2, r3, r4,
                  is0, is1, is2, is3, is4, gs0, gs1, gs2, gs3, gs4,
                  ss0, ss1, ss2, ss3, ss4):
    c = lax.axis_index("c")
    s = lax.axis_index("s")
    w = c * _NSUB + s
    ebase = w * _EPW
    pltpu.sync_copy(src_hbm.at[pl.ds(ebase, _EPW)], src_v)
    base = s * _OWN

    @pl.when(s == _NSUB - 1)
    def _():
        pltpu.sync_copy(g_hbm.at[pl.ds(c * _N + base, _OWNL)],
                        acc.at[pl.ds(base, _OWNL)])

    @pl.when(s != _NSUB - 1)
    def _():
        pltpu.sync_copy(g_hbm.at[pl.ds(c * _N + base, _OWN)],
                        acc.at[pl.ds(base, _OWN)])

    plsc.subcore_barrier()

    idx = (i0, i1, i2, i3, i4)
    rows = (r0, r1, r2, r3, r4)
    iss = (is0, is1, is2, is3, is4)
    gs = (gs0, gs1, gs2, gs3, gs4)
    ss = (ss0, ss1, ss2, ss3, ss4)

    def fetch_dst(b, j):
        pltpu.async_copy(dst_hbm.at[pl.ds(ebase + j * _K, _K)],
                         idx[b].at[0], iss[b])

    def wait_dst(b, j):
        pltpu.make_async_copy(dst_hbm.at[pl.ds(ebase + j * _K, _K)],
                              idx[b].at[0], iss[b]).wait()

    def gather(b, j):
        pltpu.async_copy(g_hbm.at[src_v.at[pl.ds(j * _K, _K)]], rows[b], gs[b])

    def wait_gather(b, j):
        pltpu.make_async_copy(g_hbm.at[src_v.at[pl.ds(j * _K, _K)]],
                              rows[b], gs[b]).wait()

    def scatter(b):
        pltpu.async_copy(rows[b], acc.at[idx[b].at[0]], ss[b], add=True)

    def wait_scatter(b):
        pltpu.make_async_copy(rows[b], acc.at[idx[b].at[0]], ss[b]).wait()

    for b in range(_NB):
        fetch_dst(b, b)
        gather(b, b)

    def pipe(jj, carry):
        j0 = jj * _NB
        for b in range(_NB):
            wait_gather(b, j0 + b)
            wait_dst(b, j0 + b)
            scatter(b)
        for b in range(_NB):
            jn = lax.rem(j0 + b + _NB, _CH)
            wait_scatter(b)
            fetch_dst(b, jn)
            gather(b, jn)
        return carry

    lax.fori_loop(0, _CH // _NB, pipe, 0)
    for b in range(_NB):
        wait_gather(b, b)
        wait_dst(b, b)
    plsc.subcore_barrier()

    @pl.when(s == _NSUB - 1)
    def _():
        pltpu.sync_copy(acc.at[pl.ds(base, _OWNL)],
                        out_hbm.at[c, pl.ds(base, _OWNL)])

    @pl.when(s != _NSUB - 1)
    def _():
        pltpu.sync_copy(acc.at[pl.ds(base, _OWN)],
                        out_hbm.at[c, pl.ds(base, _OWN)])


_RBLK = 1000


def _tc_a(x, W10, degpart):
    def body(x_ref, w_ref, dp_ref, g_ref, dinv_ref):
        dv = lax.rsqrt(dp_ref[0, :, 0:1] + dp_ref[1, :, 0:1])
        g_ref[0] = jnp.dot(x_ref[...], w_ref[...],
                           preferred_element_type=jnp.float32) * dv
        g_ref[1] = jnp.zeros((_RBLK, _H), jnp.float32)
        dinv_ref[...] = dv

    return pl.pallas_call(
        body,
        grid=(_N // _RBLK,),
        in_specs=[
            pl.BlockSpec((_RBLK, _H), lambda i: (i, 0)),
            pl.BlockSpec((_H, _H), lambda i: (0, 0)),
            pl.BlockSpec((_NSC, _RBLK, _H), lambda i: (0, i, 0)),
        ],
        out_specs=[
            pl.BlockSpec((_NSC, _RBLK, _H), lambda i: (0, i, 0)),
            pl.BlockSpec((_RBLK, 1), lambda i: (i, 0)),
        ],
        out_shape=[
            jax.ShapeDtypeStruct((_NSC, _N, _H), jnp.float32),
            jax.ShapeDtypeStruct((_N, 1), jnp.float32),
        ],
    )(x, W10, degpart)


def _tc_b(P1, dinv, b10, w1, b1, W11):
    def body(p_ref, dinv_ref, b10_ref, w1_ref, b1_ref, w11_ref, h1_ref, g2_ref):
        dv = dinv_ref[...]
        h1 = jnp.maximum((p_ref[0] + p_ref[1]) * dv + b10_ref[...], 0.0)
        h11 = jnp.maximum(
            jnp.dot(h1, w1_ref[...], preferred_element_type=jnp.float32)
            + b1_ref[...], 0.0)
        g2_ref[0] = jnp.dot(h11, w11_ref[...],
                            preferred_element_type=jnp.float32) * dv
        g2_ref[1] = jnp.zeros((_RBLK, _H), jnp.float32)
        h1_ref[...] = h1

    return pl.pallas_call(
        body,
        grid=(_N // _RBLK,),
        in_specs=[
            pl.BlockSpec((_NSC, _RBLK, _H), lambda i: (0, i, 0)),
            pl.BlockSpec((_RBLK, 1), lambda i: (i, 0)),
            pl.BlockSpec((1, _H), lambda i: (0, 0)),
            pl.BlockSpec((_H, _H), lambda i: (0, 0)),
            pl.BlockSpec((_RBLK, _H), lambda i: (i, 0)),
            pl.BlockSpec((_H, _H), lambda i: (0, 0)),
        ],
        out_specs=[
            pl.BlockSpec((_RBLK, _H), lambda i: (i, 0)),
            pl.BlockSpec((_NSC, _RBLK, _H), lambda i: (0, i, 0)),
        ],
        out_shape=[
            jax.ShapeDtypeStruct((_N, _H), jnp.float32),
            jax.ShapeDtypeStruct((_NSC, _N, _H), jnp.float32),
        ],
    )(P1, dinv, b10, w1, b1, W11)


def _tc_c(P2, dinv, b11, W12):
    def body(p_ref, dinv_ref, b11_ref, w12_ref, g3_ref):
        dv = dinv_ref[...]
        h2 = (p_ref[0] + p_ref[1]) * dv + b11_ref[...]
        g3_ref[0] = jnp.dot(h2, w12_ref[...],
                            preferred_element_type=jnp.float32) * dv
        g3_ref[1] = jnp.zeros((_RBLK, _H), jnp.float32)

    return pl.pallas_call(
        body,
        grid=(_N // _RBLK,),
        in_specs=[
            pl.BlockSpec((_NSC, _RBLK, _H), lambda i: (0, i, 0)),
            pl.BlockSpec((_RBLK, 1), lambda i: (i, 0)),
            pl.BlockSpec((1, _H), lambda i: (0, 0)),
            pl.BlockSpec((_H, _H), lambda i: (0, 0)),
        ],
        out_specs=pl.BlockSpec((_NSC, _RBLK, _H), lambda i: (0, i, 0)),
        out_shape=jax.ShapeDtypeStruct((_NSC, _N, _H), jnp.float32),
    )(P2, dinv, b11, W12)


def _tc_d(P3, dinv, b12, h1, fc0_W, fc0_b, fc1_W, fc1_b):
    def body(p_ref, dinv_ref, b12_ref, h1_ref, fc0w_ref, fc0b_ref, fc1w_ref,
             fc1b_ref, out_ref):
        h2 = jnp.maximum((p_ref[0] + p_ref[1]) * dinv_ref[...] + b12_ref[...],
                         0.0)
        h = h1_ref[...] + h2
        t = jnp.maximum(
            jnp.dot(h, fc0w_ref[...], preferred_element_type=jnp.float32)
            + fc0b_ref[...], 0.0)
        out_ref[...] = (
            jnp.dot(t, fc1w_ref[...], preferred_element_type=jnp.float32)
            + fc1b_ref[...])

    return pl.pallas_call(
        body,
        grid=(_N // _RBLK,),
        in_specs=[
            pl.BlockSpec((_NSC, _RBLK, _H), lambda i: (0, i, 0)),
            pl.BlockSpec((_RBLK, 1), lambda i: (i, 0)),
            pl.BlockSpec((1, _H), lambda i: (0, 0)),
            pl.BlockSpec((_RBLK, _H), lambda i: (i, 0)),
            pl.BlockSpec((_H, _FCW), lambda i: (0, 0)),
            pl.BlockSpec((1, _FCW), lambda i: (0, 0)),
            pl.BlockSpec((_FCW, _NCLS), lambda i: (0, 0)),
            pl.BlockSpec((1, _NCLS), lambda i: (0, 0)),
        ],
        out_specs=pl.BlockSpec((_RBLK, _NCLS), lambda i: (i, 0)),
        out_shape=jax.ShapeDtypeStruct((_N, _NCLS), jnp.float32),
    )(P3, dinv, b12, h1, fc0_W, fc0_b, fc1_W, fc1_b)


def kernel(x, edge_index, W10, b10, W11, b11, W12, b12, w1, b1, fc0_W, fc0_b,
           fc1_W, fc1_b):
    src = edge_index[0]
    dst = edge_index[1]

    ones2 = jnp.concatenate([jnp.ones((_N, _H), jnp.float32),
                             jnp.zeros((_N, _H), jnp.float32)])
    degpart = _sc_aggregate(ones2, src, dst)
    g1, dinv = _tc_a(x, W10, degpart)
    P1 = _sc_aggregate(g1.reshape(_NSC * _N, _H), src, dst)
    h1, g2 = _tc_b(P1, dinv, b10.reshape(1, -1), w1, b1, W11)
    P2 = _sc_aggregate(g2.reshape(_NSC * _N, _H), src, dst)
    g3 = _tc_c(P2, dinv, b11.reshape(1, -1), W12)
    P3 = _sc_aggregate(g3.reshape(_NSC * _N, _H), src, dst)
    out = _tc_d(P3, dinv, b12.reshape(1, -1), h1, fc0_W, fc0_b.reshape(1, -1),
                fc1_W, fc1_b.reshape(1, -1))
    return out


# speedup vs baseline: 21.7465x; 1.0048x over previous
import functools

import jax
import jax.numpy as jnp
from jax import lax
from jax.experimental import pallas as pl
from jax.experimental.pallas import tpu as pltpu
from jax.experimental.pallas import tpu_sc as plsc

_N = 10000
_E = 320000
_H = 128
_FCW = 1024
_NCLS = 16

_NSC = 2
_NSUB = 16
_NW = _NSC * _NSUB
_EPW = _E // _NW
_K = 64
_CH = _EPW // _K
_KT = _EPW - _CH * _K
_NB = 4
_OWN = 624
_OWNL = _N - (_NSUB - 1) * _OWN

_sc_mesh = plsc.VectorSubcoreMesh(
    core_axis_name="c", subcore_axis_name="s", num_cores=_NSC, num_subcores=_NSUB
)


@functools.partial(
    pl.kernel,
    out_type=jax.ShapeDtypeStruct((_NSC, _N, _H), jnp.float32),
    mesh=_sc_mesh,
    scratch_types=[
        pltpu.VMEM_SHARED((_N, _H), jnp.float32),
        pltpu.VMEM((_EPW,), jnp.int32),
        pltpu.VMEM((1, _K), jnp.int32),
        pltpu.VMEM((1, _K), jnp.int32),
        pltpu.VMEM((1, _K), jnp.int32),
        pltpu.VMEM((1, _K), jnp.int32),
        pltpu.VMEM((1, _KT), jnp.int32),
        pltpu.VMEM((_K, _H), jnp.float32),
        pltpu.VMEM((_K, _H), jnp.float32),
        pltpu.VMEM((_K, _H), jnp.float32),
        pltpu.VMEM((_K, _H), jnp.float32),
        pltpu.SemaphoreType.DMA,
        pltpu.SemaphoreType.DMA,
        pltpu.SemaphoreType.DMA,
        pltpu.SemaphoreType.DMA,
        pltpu.SemaphoreType.DMA,
        pltpu.SemaphoreType.DMA,
        pltpu.SemaphoreType.DMA,
        pltpu.SemaphoreType.DMA,
        pltpu.SemaphoreType.DMA,
        pltpu.SemaphoreType.DMA,
        pltpu.SemaphoreType.DMA,
        pltpu.SemaphoreType.DMA,
    ],
)
def _sc_aggregate(g_hbm, src_hbm, dst_hbm, out_hbm, acc, src_v,
                  i0, i1, i2, i3, it, r0, r1, r2, r3,
                  is0, is1, is2, is3, gs0, gs1, gs2, gs3,
                  ss0, ss1, ss2, ss3):
    c = lax.axis_index("c")
    s = lax.axis_index("s")
    w = c * _NSUB + s
    ebase = w * _EPW
    pltpu.sync_copy(src_hbm.at[pl.ds(ebase, _EPW)], src_v)
    base = s * _OWN

    @pl.when(s == _NSUB - 1)
    def _():
        pltpu.sync_copy(g_hbm.at[pl.ds(c * _N + base, _OWNL)],
                        acc.at[pl.ds(base, _OWNL)])

    @pl.when(s != _NSUB - 1)
    def _():
        pltpu.sync_copy(g_hbm.at[pl.ds(c * _N + base, _OWN)],
                        acc.at[pl.ds(base, _OWN)])

    plsc.subcore_barrier()

    idx = (i0, i1, i2, i3)
    rows = (r0, r1, r2, r3)
    iss = (is0, is1, is2, is3)
    gs = (gs0, gs1, gs2, gs3)
    ss = (ss0, ss1, ss2, ss3)

    def fetch_dst(b, j):
        pltpu.async_copy(dst_hbm.at[pl.ds(ebase + j * _K, _K)],
                         idx[b].at[0], iss[b])

    def wait_dst(b, j):
        pltpu.make_async_copy(dst_hbm.at[pl.ds(ebase + j * _K, _K)],
                              idx[b].at[0], iss[b]).wait()

    def gather(b, j):
        pltpu.async_copy(g_hbm.at[src_v.at[pl.ds(j * _K, _K)]], rows[b], gs[b])

    def wait_gather(b, j):
        pltpu.make_async_copy(g_hbm.at[src_v.at[pl.ds(j * _K, _K)]],
                              rows[b], gs[b]).wait()

    def scatter(b):
        pltpu.async_copy(rows[b], acc.at[idx[b].at[0]], ss[b], add=True)

    def wait_scatter(b):
        pltpu.make_async_copy(rows[b], acc.at[idx[b].at[0]], ss[b]).wait()

    for b in range(_NB):
        fetch_dst(b, b)
        gather(b, b)

    def pipe(jj, carry):
        j0 = jj * _NB
        for b in range(_NB):
            wait_gather(b, j0 + b)
            wait_dst(b, j0 + b)
            scatter(b)
        for b in range(_NB):
            jn = lax.rem(j0 + b + _NB, _CH)
            wait_scatter(b)
            fetch_dst(b, jn)
            gather(b, jn)
        return carry

    lax.fori_loop(0, _CH // _NB, pipe, 0)
    for b in range(_NB):
        wait_gather(b, b)
        wait_dst(b, b)

    toff = _CH * _K
    pltpu.async_copy(dst_hbm.at[pl.ds(ebase + toff, _KT)], it.at[0], is0)
    tg = pltpu.async_copy(g_hbm.at[src_v.at[pl.ds(toff, _KT)]],
                          r0.at[pl.ds(0, _KT)], gs0)
    pltpu.make_async_copy(dst_hbm.at[pl.ds(ebase + toff, _KT)],
                          it.at[0], is0).wait()
    tg.wait()
    pltpu.sync_copy(r0.at[pl.ds(0, _KT)], acc.at[it.at[0]], add=True)
    plsc.subcore_barrier()

    @pl.when(s == _NSUB - 1)
    def _():
        pltpu.sync_copy(acc.at[pl.ds(base, _OWNL)],
                        out_hbm.at[c, pl.ds(base, _OWNL)])

    @pl.when(s != _NSUB - 1)
    def _():
        pltpu.sync_copy(acc.at[pl.ds(base, _OWN)],
                        out_hbm.at[c, pl.ds(base, _OWN)])


_RBLK = 1000


def _tc_a(x, W10, degpart):
    def body(x_ref, w_ref, dp_ref, g_ref, dinv_ref):
        dv = lax.rsqrt(dp_ref[0, :, 0:1] + dp_ref[1, :, 0:1])
        g_ref[0] = jnp.dot(x_ref[...], w_ref[...],
                           preferred_element_type=jnp.float32) * dv
        g_ref[1] = jnp.zeros((_RBLK, _H), jnp.float32)
        dinv_ref[...] = dv

    return pl.pallas_call(
        body,
        grid=(_N // _RBLK,),
        in_specs=[
            pl.BlockSpec((_RBLK, _H), lambda i: (i, 0)),
            pl.BlockSpec((_H, _H), lambda i: (0, 0)),
            pl.BlockSpec((_NSC, _RBLK, _H), lambda i: (0, i, 0)),
        ],
        out_specs=[
            pl.BlockSpec((_NSC, _RBLK, _H), lambda i: (0, i, 0)),
            pl.BlockSpec((_RBLK, 1), lambda i: (i, 0)),
        ],
        out_shape=[
            jax.ShapeDtypeStruct((_NSC, _N, _H), jnp.float32),
            jax.ShapeDtypeStruct((_N, 1), jnp.float32),
        ],
    )(x, W10, degpart)


def _tc_b(P1, dinv, b10, w1, b1, W11):
    def body(p_ref, dinv_ref, b10_ref, w1_ref, b1_ref, w11_ref, h1_ref, g2_ref):
        dv = dinv_ref[...]
        h1 = jnp.maximum((p_ref[0] + p_ref[1]) * dv + b10_ref[...], 0.0)
        h11 = jnp.maximum(
            jnp.dot(h1, w1_ref[...], preferred_element_type=jnp.float32)
            + b1_ref[...], 0.0)
        g2_ref[0] = jnp.dot(h11, w11_ref[...],
                            preferred_element_type=jnp.float32) * dv
        g2_ref[1] = jnp.zeros((_RBLK, _H), jnp.float32)
        h1_ref[...] = h1

    return pl.pallas_call(
        body,
        grid=(_N // _RBLK,),
        in_specs=[
            pl.BlockSpec((_NSC, _RBLK, _H), lambda i: (0, i, 0)),
            pl.BlockSpec((_RBLK, 1), lambda i: (i, 0)),
            pl.BlockSpec((1, _H), lambda i: (0, 0)),
            pl.BlockSpec((_H, _H), lambda i: (0, 0)),
            pl.BlockSpec((_RBLK, _H), lambda i: (i, 0)),
            pl.BlockSpec((_H, _H), lambda i: (0, 0)),
        ],
        out_specs=[
            pl.BlockSpec((_RBLK, _H), lambda i: (i, 0)),
            pl.BlockSpec((_NSC, _RBLK, _H), lambda i: (0, i, 0)),
        ],
        out_shape=[
            jax.ShapeDtypeStruct((_N, _H), jnp.float32),
            jax.ShapeDtypeStruct((_NSC, _N, _H), jnp.float32),
        ],
    )(P1, dinv, b10, w1, b1, W11)


def _tc_c(P2, dinv, b11, W12):
    def body(p_ref, dinv_ref, b11_ref, w12_ref, g3_ref):
        dv = dinv_ref[...]
        h2 = (p_ref[0] + p_ref[1]) * dv + b11_ref[...]
        g3_ref[0] = jnp.dot(h2, w12_ref[...],
                            preferred_element_type=jnp.float32) * dv
        g3_ref[1] = jnp.zeros((_RBLK, _H), jnp.float32)

    return pl.pallas_call(
        body,
        grid=(_N // _RBLK,),
        in_specs=[
            pl.BlockSpec((_NSC, _RBLK, _H), lambda i: (0, i, 0)),
            pl.BlockSpec((_RBLK, 1), lambda i: (i, 0)),
            pl.BlockSpec((1, _H), lambda i: (0, 0)),
            pl.BlockSpec((_H, _H), lambda i: (0, 0)),
        ],
        out_specs=pl.BlockSpec((_NSC, _RBLK, _H), lambda i: (0, i, 0)),
        out_shape=jax.ShapeDtypeStruct((_NSC, _N, _H), jnp.float32),
    )(P2, dinv, b11, W12)


def _tc_d(P3, dinv, b12, h1, fc0_W, fc0_b, fc1_W, fc1_b):
    def body(p_ref, dinv_ref, b12_ref, h1_ref, fc0w_ref, fc0b_ref, fc1w_ref,
             fc1b_ref, out_ref):
        h2 = jnp.maximum((p_ref[0] + p_ref[1]) * dinv_ref[...] + b12_ref[...],
                         0.0)
        h = h1_ref[...] + h2
        t = jnp.maximum(
            jnp.dot(h, fc0w_ref[...], preferred_element_type=jnp.float32)
            + fc0b_ref[...], 0.0)
        out_ref[...] = (
            jnp.dot(t, fc1w_ref[...], preferred_element_type=jnp.float32)
            + fc1b_ref[...])

    return pl.pallas_call(
        body,
        grid=(_N // _RBLK,),
        in_specs=[
            pl.BlockSpec((_NSC, _RBLK, _H), lambda i: (0, i, 0)),
            pl.BlockSpec((_RBLK, 1), lambda i: (i, 0)),
            pl.BlockSpec((1, _H), lambda i: (0, 0)),
            pl.BlockSpec((_RBLK, _H), lambda i: (i, 0)),
            pl.BlockSpec((_H, _FCW), lambda i: (0, 0)),
            pl.BlockSpec((1, _FCW), lambda i: (0, 0)),
            pl.BlockSpec((_FCW, _NCLS), lambda i: (0, 0)),
            pl.BlockSpec((1, _NCLS), lambda i: (0, 0)),
        ],
        out_specs=pl.BlockSpec((_RBLK, _NCLS), lambda i: (i, 0)),
        out_shape=jax.ShapeDtypeStruct((_N, _NCLS), jnp.float32),
    )(P3, dinv, b12, h1, fc0_W, fc0_b, fc1_W, fc1_b)


def kernel(x, edge_index, W10, b10, W11, b11, W12, b12, w1, b1, fc0_W, fc0_b,
           fc1_W, fc1_b):
    src = edge_index[0]
    dst = edge_index[1]

    ones2 = jnp.concatenate([jnp.ones((_N, _H), jnp.float32),
                             jnp.zeros((_N, _H), jnp.float32)])
    degpart = _sc_aggregate(ones2, src, dst)
    g1, dinv = _tc_a(x, W10, degpart)
    P1 = _sc_aggregate(g1.reshape(_NSC * _N, _H), src, dst)
    h1, g2 = _tc_b(P1, dinv, b10.reshape(1, -1), w1, b1, W11)
    P2 = _sc_aggregate(g2.reshape(_NSC * _N, _H), src, dst)
    g3 = _tc_c(P2, dinv, b11.reshape(1, -1), W12)
    P3 = _sc_aggregate(g3.reshape(_NSC * _N, _H), src, dst)
    out = _tc_d(P3, dinv, b12.reshape(1, -1), h1, fc0_W, fc0_b.reshape(1, -1),
                fc1_W, fc1_b.reshape(1, -1))
    return out


# speedup vs baseline: 21.7662x; 1.0009x over previous
import functools

import jax
import jax.numpy as jnp
from jax import lax
from jax.experimental import pallas as pl
from jax.experimental.pallas import tpu as pltpu
from jax.experimental.pallas import tpu_sc as plsc

_N = 10000
_E = 320000
_H = 128
_FCW = 1024
_NCLS = 16

_NSC = 2
_NSUB = 16
_NW = _NSC * _NSUB
_EPW = _E // _NW
_K = 64
_CH = _EPW // _K
_KT = _EPW - _CH * _K
_NB = 4
_OWN = 624
_OWNL = _N - (_NSUB - 1) * _OWN

_sc_mesh = plsc.VectorSubcoreMesh(
    core_axis_name="c", subcore_axis_name="s", num_cores=_NSC, num_subcores=_NSUB
)


@functools.partial(
    pl.kernel,
    out_type=jax.ShapeDtypeStruct((_NSC, _N, _H), jnp.float32),
    mesh=_sc_mesh,
    scratch_types=[
        pltpu.VMEM_SHARED((_N, _H), jnp.float32),
        pltpu.VMEM((_EPW,), jnp.int32),
        pltpu.VMEM((1, _K), jnp.int32),
        pltpu.VMEM((1, _K), jnp.int32),
        pltpu.VMEM((1, _K), jnp.int32),
        pltpu.VMEM((1, _K), jnp.int32),
        pltpu.VMEM((1, _KT), jnp.int32),
        pltpu.VMEM((_K, _H), jnp.float32),
        pltpu.VMEM((_K, _H), jnp.float32),
        pltpu.VMEM((_K, _H), jnp.float32),
        pltpu.VMEM((_K, _H), jnp.float32),
        pltpu.SemaphoreType.DMA,
        pltpu.SemaphoreType.DMA,
        pltpu.SemaphoreType.DMA,
        pltpu.SemaphoreType.DMA,
        pltpu.SemaphoreType.DMA,
        pltpu.SemaphoreType.DMA,
        pltpu.SemaphoreType.DMA,
        pltpu.SemaphoreType.DMA,
        pltpu.SemaphoreType.DMA,
        pltpu.SemaphoreType.DMA,
        pltpu.SemaphoreType.DMA,
        pltpu.SemaphoreType.DMA,
    ],
)
def _sc_aggregate(g_hbm, src_hbm, dst_hbm, out_hbm, acc, src_v,
                  i0, i1, i2, i3, it, r0, r1, r2, r3,
                  is0, is1, is2, is3, gs0, gs1, gs2, gs3,
                  ss0, ss1, ss2, ss3):
    c = lax.axis_index("c")
    s = lax.axis_index("s")
    w = c * _NSUB + s
    ebase = w * _EPW
    pltpu.sync_copy(src_hbm.at[pl.ds(ebase, _EPW)], src_v)
    base = s * _OWN

    @pl.when(s == _NSUB - 1)
    def _():
        pltpu.sync_copy(g_hbm.at[pl.ds(c * _N + base, _OWNL)],
                        acc.at[pl.ds(base, _OWNL)])

    @pl.when(s != _NSUB - 1)
    def _():
        pltpu.sync_copy(g_hbm.at[pl.ds(c * _N + base, _OWN)],
                        acc.at[pl.ds(base, _OWN)])

    plsc.subcore_barrier()

    idx = (i0, i1, i2, i3)
    rows = (r0, r1, r2, r3)
    iss = (is0, is1, is2, is3)
    gs = (gs0, gs1, gs2, gs3)
    ss = (ss0, ss1, ss2, ss3)

    def fetch_dst(b, j):
        pltpu.async_copy(dst_hbm.at[pl.ds(ebase + j * _K, _K)],
                         idx[b].at[0], iss[b])

    def wait_dst(b, j):
        pltpu.make_async_copy(dst_hbm.at[pl.ds(ebase + j * _K, _K)],
                              idx[b].at[0], iss[b]).wait()

    def gather(b, j):
        pltpu.async_copy(g_hbm.at[src_v.at[pl.ds(j * _K, _K)]], rows[b], gs[b])

    def wait_gather(b, j):
        pltpu.make_async_copy(g_hbm.at[src_v.at[pl.ds(j * _K, _K)]],
                              rows[b], gs[b]).wait()

    def scatter(b):
        pltpu.async_copy(rows[b], acc.at[idx[b].at[0]], ss[b], add=True)

    def wait_scatter(b):
        pltpu.make_async_copy(rows[b], acc.at[idx[b].at[0]], ss[b]).wait()

    for b in range(_NB):
        fetch_dst(b, b)
        gather(b, b)

    def pipe(jj, carry):
        j0 = jj * _NB
        for b in range(_NB):
            wait_gather(b, j0 + b)
            wait_dst(b, j0 + b)
            scatter(b)
        for b in range(_NB):
            jn = lax.rem(j0 + b + _NB, _CH)
            wait_scatter(b)
            fetch_dst(b, jn)
            gather(b, jn)
        return carry

    lax.fori_loop(0, _CH // _NB, pipe, 0)
    for b in range(_NB):
        wait_gather(b, b)
        wait_dst(b, b)

    toff = _CH * _K
    pltpu.async_copy(dst_hbm.at[pl.ds(ebase + toff, _KT)], it.at[0], is0)
    tg = pltpu.async_copy(g_hbm.at[src_v.at[pl.ds(toff, _KT)]],
                          r0.at[pl.ds(0, _KT)], gs0)
    pltpu.make_async_copy(dst_hbm.at[pl.ds(ebase + toff, _KT)],
                          it.at[0], is0).wait()
    tg.wait()
    pltpu.sync_copy(r0.at[pl.ds(0, _KT)], acc.at[it.at[0]], add=True)
    plsc.subcore_barrier()

    @pl.when(s == _NSUB - 1)
    def _():
        pltpu.sync_copy(acc.at[pl.ds(base, _OWNL)],
                        out_hbm.at[c, pl.ds(base, _OWNL)])

    @pl.when(s != _NSUB - 1)
    def _():
        pltpu.sync_copy(acc.at[pl.ds(base, _OWN)],
                        out_hbm.at[c, pl.ds(base, _OWN)])


_RBLK = 1000


def _tc_a0(x, W10):
    def body(x_ref, w_ref, h_ref):
        h_ref[...] = jnp.dot(x_ref[...], w_ref[...],
                             preferred_element_type=jnp.float32)

    return pl.pallas_call(
        body,
        grid=(_N // _RBLK,),
        in_specs=[
            pl.BlockSpec((_RBLK, _H), lambda i: (i, 0)),
            pl.BlockSpec((_H, _H), lambda i: (0, 0)),
        ],
        out_specs=pl.BlockSpec((_RBLK, _H), lambda i: (i, 0)),
        out_shape=jax.ShapeDtypeStruct((_N, _H), jnp.float32),
    )(x, W10)


def _tc_a(h10, degpart):
    def body(h_ref, dp_ref, g_ref, dinv_ref):
        dv = lax.rsqrt(dp_ref[0, :, 0:1] + dp_ref[1, :, 0:1])
        g_ref[0] = h_ref[...] * dv
        g_ref[1] = jnp.zeros((_RBLK, _H), jnp.float32)
        dinv_ref[...] = dv

    return pl.pallas_call(
        body,
        grid=(_N // _RBLK,),
        in_specs=[
            pl.BlockSpec((_RBLK, _H), lambda i: (i, 0)),
            pl.BlockSpec((_NSC, _RBLK, _H), lambda i: (0, i, 0)),
        ],
        out_specs=[
            pl.BlockSpec((_NSC, _RBLK, _H), lambda i: (0, i, 0)),
            pl.BlockSpec((_RBLK, 1), lambda i: (i, 0)),
        ],
        out_shape=[
            jax.ShapeDtypeStruct((_NSC, _N, _H), jnp.float32),
            jax.ShapeDtypeStruct((_N, 1), jnp.float32),
        ],
    )(h10, degpart)


def _tc_b(P1, dinv, b10, w1, b1, W11):
    def body(p_ref, dinv_ref, b10_ref, w1_ref, b1_ref, w11_ref, h1_ref, g2_ref):
        dv = dinv_ref[...]
        h1 = jnp.maximum((p_ref[0] + p_ref[1]) * dv + b10_ref[...], 0.0)
        h11 = jnp.maximum(
            jnp.dot(h1, w1_ref[...], preferred_element_type=jnp.float32)
            + b1_ref[...], 0.0)
        g2_ref[0] = jnp.dot(h11, w11_ref[...],
                            preferred_element_type=jnp.float32) * dv
        g2_ref[1] = jnp.zeros((_RBLK, _H), jnp.float32)
        h1_ref[...] = h1

    return pl.pallas_call(
        body,
        grid=(_N // _RBLK,),
        in_specs=[
            pl.BlockSpec((_NSC, _RBLK, _H), lambda i: (0, i, 0)),
            pl.BlockSpec((_RBLK, 1), lambda i: (i, 0)),
            pl.BlockSpec((1, _H), lambda i: (0, 0)),
            pl.BlockSpec((_H, _H), lambda i: (0, 0)),
            pl.BlockSpec((_RBLK, _H), lambda i: (i, 0)),
            pl.BlockSpec((_H, _H), lambda i: (0, 0)),
        ],
        out_specs=[
            pl.BlockSpec((_RBLK, _H), lambda i: (i, 0)),
            pl.BlockSpec((_NSC, _RBLK, _H), lambda i: (0, i, 0)),
        ],
        out_shape=[
            jax.ShapeDtypeStruct((_N, _H), jnp.float32),
            jax.ShapeDtypeStruct((_NSC, _N, _H), jnp.float32),
        ],
    )(P1, dinv, b10, w1, b1, W11)


def _tc_c(P2, dinv, b11, W12):
    def body(p_ref, dinv_ref, b11_ref, w12_ref, g3_ref):
        dv = dinv_ref[...]
        h2 = (p_ref[0] + p_ref[1]) * dv + b11_ref[...]
        g3_ref[0] = jnp.dot(h2, w12_ref[...],
                            preferred_element_type=jnp.float32) * dv
        g3_ref[1] = jnp.zeros((_RBLK, _H), jnp.float32)

    return pl.pallas_call(
        body,
        grid=(_N // _RBLK,),
        in_specs=[
            pl.BlockSpec((_NSC, _RBLK, _H), lambda i: (0, i, 0)),
            pl.BlockSpec((_RBLK, 1), lambda i: (i, 0)),
            pl.BlockSpec((1, _H), lambda i: (0, 0)),
            pl.BlockSpec((_H, _H), lambda i: (0, 0)),
        ],
        out_specs=pl.BlockSpec((_NSC, _RBLK, _H), lambda i: (0, i, 0)),
        out_shape=jax.ShapeDtypeStruct((_NSC, _N, _H), jnp.float32),
    )(P2, dinv, b11, W12)


def _tc_d(P3, dinv, b12, h1, fc0_W, fc0_b, fc1_W, fc1_b):
    def body(p_ref, dinv_ref, b12_ref, h1_ref, fc0w_ref, fc0b_ref, fc1w_ref,
             fc1b_ref, out_ref):
        h2 = jnp.maximum((p_ref[0] + p_ref[1]) * dinv_ref[...] + b12_ref[...],
                         0.0)
        h = h1_ref[...] + h2
        t = jnp.maximum(
            jnp.dot(h, fc0w_ref[...], preferred_element_type=jnp.float32)
            + fc0b_ref[...], 0.0)
        out_ref[...] = (
            jnp.dot(t, fc1w_ref[...], preferred_element_type=jnp.float32)
            + fc1b_ref[...])

    return pl.pallas_call(
        body,
        grid=(_N // _RBLK,),
        in_specs=[
            pl.BlockSpec((_NSC, _RBLK, _H), lambda i: (0, i, 0)),
            pl.BlockSpec((_RBLK, 1), lambda i: (i, 0)),
            pl.BlockSpec((1, _H), lambda i: (0, 0)),
            pl.BlockSpec((_RBLK, _H), lambda i: (i, 0)),
            pl.BlockSpec((_H, _FCW), lambda i: (0, 0)),
            pl.BlockSpec((1, _FCW), lambda i: (0, 0)),
            pl.BlockSpec((_FCW, _NCLS), lambda i: (0, 0)),
            pl.BlockSpec((1, _NCLS), lambda i: (0, 0)),
        ],
        out_specs=pl.BlockSpec((_RBLK, _NCLS), lambda i: (i, 0)),
        out_shape=jax.ShapeDtypeStruct((_N, _NCLS), jnp.float32),
    )(P3, dinv, b12, h1, fc0_W, fc0_b, fc1_W, fc1_b)


def kernel(x, edge_index, W10, b10, W11, b11, W12, b12, w1, b1, fc0_W, fc0_b,
           fc1_W, fc1_b):
    src = edge_index[0]
    dst = edge_index[1]

    ones2 = jnp.concatenate([jnp.ones((_N, _H), jnp.float32),
                             jnp.zeros((_N, _H), jnp.float32)])
    degpart = _sc_aggregate(ones2, src, dst)
    h10 = _tc_a0(x, W10)
    g1, dinv = _tc_a(h10, degpart)
    P1 = _sc_aggregate(g1.reshape(_NSC * _N, _H), src, dst)
    h1, g2 = _tc_b(P1, dinv, b10.reshape(1, -1), w1, b1, W11)
    P2 = _sc_aggregate(g2.reshape(_NSC * _N, _H), src, dst)
    g3 = _tc_c(P2, dinv, b11.reshape(1, -1), W12)
    P3 = _sc_aggregate(g3.reshape(_NSC * _N, _H), src, dst)
    out = _tc_d(P3, dinv, b12.reshape(1, -1), h1, fc0_W, fc0_b.reshape(1, -1),
                fc1_W, fc1_b.reshape(1, -1))
    return out
